# Initial kernel scaffold; baseline (speedup 1.0000x reference)
import functools

import jax
import jax.numpy as jnp
from jax import lax
from jax.experimental import pallas as pl
from jax.experimental.pallas import tpu as pltpu
from jax.experimental.pallas import tpu_sc as plsc

N = 10000
E = 320000
H = 128
NG = 64
EPS = 1e-5

NC = 2
NS = 16
NW = NC * NS
CH = 128
CPT = 79
EPT = CPT * CH
EPAD = NW * EPT

DUMP = N
ACC_ROWS = 10016
ROWS_INIT = ACC_ROWS // NS
ROWS_OUT = N // NS

DEG_LEN = 10240
DEG_PT = DEG_LEN // NS


def _mesh():
    return plsc.VectorSubcoreMesh(
        core_axis_name="c", subcore_axis_name="s", num_cores=NC,
        num_subcores=NS)


def _deg_body(dst_hbm, out_hbm, idx_v, ones_v, zbuf_v, acc_sh):
    c = lax.axis_index("c")
    s = lax.axis_index("s")
    wid = s * NC + c
    for k in range(CH // 16):
        ones_v[pl.ds(k * 16, 16)] = jnp.ones((16,), jnp.float32)
    for k in range(DEG_PT // 16):
        zbuf_v[pl.ds(k * 16, 16)] = jnp.zeros((16,), jnp.float32)
    pltpu.sync_copy(zbuf_v, acc_sh.at[pl.ds(s * DEG_PT, DEG_PT)])
    pltpu.sync_copy(dst_hbm.at[wid], idx_v)
    plsc.subcore_barrier()

    def body(j, carry):
        pltpu.sync_copy(ones_v, acc_sh.at[idx_v.at[j]], add=True)
        return carry

    lax.fori_loop(0, CPT, body, 0)
    plsc.subcore_barrier()
    pltpu.sync_copy(acc_sh.at[pl.ds(s * DEG_PT, DEG_PT)],
                    out_hbm.at[c, pl.ds(s * DEG_PT, DEG_PT)])


_deg_call = pl.kernel(
    _deg_body,
    out_type=jax.ShapeDtypeStruct((NC, DEG_LEN), jnp.float32),
    mesh=_mesh(),
    scratch_types=[
        pltpu.VMEM((CPT, CH), jnp.int32),
        pltpu.VMEM((CH,), jnp.float32),
        pltpu.VMEM((DEG_PT,), jnp.float32),
        pltpu.VMEM_SHARED((DEG_LEN,), jnp.float32),
    ],
)


def _agg_body(hs_hbm, src_hbm, dst_hbm, out_hbm, srcv, dstv, rows_v, acc_sh,
              sem):
    c = lax.axis_index("c")
    s = lax.axis_index("s")
    wid = s * NC + c

    def zrow(j, carry):
        for k in range(H // 16):
            rows_v[j, pl.ds(k * 16, 16)] = jnp.zeros((16,), jnp.float32)
        return carry

    lax.fori_loop(0, CH, zrow, 0)
    base = s * ROWS_INIT
    off = 0
    for nblk in (128, 128, 128, 128, ROWS_INIT - 4 * 128):
        pltpu.sync_copy(rows_v.at[pl.ds(0, nblk)],
                        acc_sh.at[pl.ds(base + off, nblk)])
        off += nblk
    pltpu.sync_copy(src_hbm.at[wid], srcv)
    pltpu.sync_copy(dst_hbm.at[wid], dstv)
    plsc.subcore_barrier()

    def body(j, carry):
        pltpu.async_copy(hs_hbm.at[srcv.at[j]], rows_v, sem).wait()
        pltpu.sync_copy(rows_v, acc_sh.at[dstv.at[j]], add=True)
        return carry

    lax.fori_loop(0, CPT, body, 0)
    plsc.subcore_barrier()
    pltpu.sync_copy(acc_sh.at[pl.ds(s * ROWS_OUT, ROWS_OUT)],
                    out_hbm.at[c, pl.ds(s * ROWS_OUT, ROWS_OUT)])


_agg_call = pl.kernel(
    _agg_body,
    out_type=jax.ShapeDtypeStruct((NC, N, H), jnp.float32),
    mesh=_mesh(),
    scratch_types=[
        pltpu.VMEM((CPT, CH), jnp.int32),
        pltpu.VMEM((CPT, CH), jnp.int32),
        pltpu.VMEM((CH, H), jnp.float32),
        pltpu.VMEM_SHARED((ACC_ROWS, H), jnp.float32),
        pltpu.SemaphoreType.DMA,
    ],
)


def _prep_body(degp_ref, x_ref, w_ref, hs_ref, dinv_ref):
    deg = degp_ref[0] + degp_ref[1] + 1.0
    dinv = lax.rsqrt(deg)
    dinv_ref[...] = dinv
    hs_ref[...] = dinv * jnp.dot(x_ref[...], w_ref[...],
                                 preferred_element_type=jnp.float32)


def _mid_body(accs_ref, hsp_ref, dinv_ref, b_ref, g_ref, be_ref, m_ref,
              v_ref, w_ref, out_ref):
    dinv = dinv_ref[...]
    agg = dinv * (accs_ref[0] + accs_ref[1] + hsp_ref[...]) + b_ref[...]
    sc = g_ref[...] * lax.rsqrt(v_ref[...] + EPS)
    h = jnp.maximum(agg * sc + (be_ref[...] - m_ref[...] * sc), 0.0)
    out_ref[...] = dinv * jnp.dot(h, w_ref[...],
                                  preferred_element_type=jnp.float32)


def _final_body(accs_ref, hsp_ref, dinv_ref, b_ref, g_ref, be_ref, m_ref,
                v_ref, batch_ref, fw1_ref, fb1_ref, fw2_ref, fb2_ref,
                out_ref):
    dinv = dinv_ref[...]
    agg = dinv * (accs_ref[0] + accs_ref[1] + hsp_ref[...]) + b_ref[...]
    sc = g_ref[...] * lax.rsqrt(v_ref[...] + EPS)
    h = jnp.maximum(agg * sc + (be_ref[...] - m_ref[...] * sc), 0.0)
    gid = lax.broadcasted_iota(jnp.int32, (NG, N), 0)
    mask = jnp.where(gid == batch_ref[...], 1.0, 0.0)
    sums = jnp.dot(mask, h, preferred_element_type=jnp.float32)
    cnt = jnp.sum(mask, axis=1, keepdims=True)
    pooled = sums / jnp.maximum(cnt, 1.0)
    r = jnp.maximum(
        jnp.dot(pooled, fw1_ref[...], preferred_element_type=jnp.float32)
        + fb1_ref[...], 0.0)
    o = (jnp.dot(r, fw2_ref[...], preferred_element_type=jnp.float32)
         + fb2_ref[...])
    mx = jnp.max(o, axis=1, keepdims=True)
    ex = jnp.exp(o - mx)
    out_ref[...] = o - mx - jnp.log(jnp.sum(ex, axis=1, keepdims=True))


def _tc(body, out_shape, *args):
    return pl.pallas_call(body, out_shape=out_shape)(*args)


def kernel(x, edge_index, batch, W1, b1, W2, b2, W3, b3, g1, be1, m1, v1,
           g2, be2, m2, v2, g3, be3, m3, v3, fw1, fb1, fw2, fb2):
    f32 = jnp.float32
    src = edge_index[0]
    dst = edge_index[1]
    pad = EPAD - E
    src3 = jnp.concatenate([src, jnp.zeros((pad,), jnp.int32)]).reshape(
        NW, CPT, CH)
    dst3 = jnp.concatenate([dst, jnp.full((pad,), DUMP, jnp.int32)]).reshape(
        NW, CPT, CH)

    deg_p = _deg_call(dst3)
    degp = deg_p[:, :N].reshape(NC, N, 1)

    row = lambda a: a.reshape(1, -1)
    hs1, dinv = _tc(
        _prep_body,
        (jax.ShapeDtypeStruct((N, H), f32), jax.ShapeDtypeStruct((N, 1), f32)),
        degp, x, W1)

    accs1 = _agg_call(hs1, src3, dst3)
    hs2 = _tc(_mid_body, jax.ShapeDtypeStruct((N, H), f32),
              accs1, hs1, dinv, row(b1), row(g1), row(be1), row(m1), row(v1),
              W2)
    accs2 = _agg_call(hs2, src3, dst3)
    hs3 = _tc(_mid_body, jax.ShapeDtypeStruct((N, H), f32),
              accs2, hs2, dinv, row(b2), row(g2), row(be2), row(m2), row(v2),
              W3)
    accs3 = _agg_call(hs3, src3, dst3)
    out = _tc(_final_body, jax.ShapeDtypeStruct((NG, 2), f32),
              accs3, hs3, dinv, row(b3), row(g3), row(be3), row(m3), row(v3),
              row(batch.astype(jnp.int32)), fw1, row(fb1), fw2, row(fb2))
    return out


# speedup vs baseline: 10.7720x; 10.7720x over previous
import functools

import jax
import jax.numpy as jnp
from jax import lax
from jax.experimental import pallas as pl
from jax.experimental.pallas import tpu as pltpu
from jax.experimental.pallas import tpu_sc as plsc

N = 10000
E = 320000
H = 128
NG = 64
EPS = 1e-5

NC = 2
NS = 16
NW = NC * NS
CH = 128
CPT = 79
EPT = CPT * CH
EPAD = NW * EPT

DUMP = N
ACC_ROWS = 10112
ROWS_INIT = ACC_ROWS // NS
ROWS_OUT = 624

DEG_LEN = 10240
DEG_PT = DEG_LEN // NS


def _mesh():
    return plsc.VectorSubcoreMesh(
        core_axis_name="c", subcore_axis_name="s", num_cores=NC,
        num_subcores=NS)


def _deg_body(dst_hbm, out_hbm, idx_v, ones_v, zbuf_v, acc_sh):
    c = lax.axis_index("c")
    s = lax.axis_index("s")
    wid = s * NC + c
    for k in range(CH // 16):
        ones_v[pl.ds(k * 16, 16)] = jnp.ones((16,), jnp.float32)
    for k in range(DEG_PT // 16):
        zbuf_v[pl.ds(k * 16, 16)] = jnp.zeros((16,), jnp.float32)
    pltpu.sync_copy(zbuf_v, acc_sh.at[pl.ds(s * DEG_PT, DEG_PT)])
    pltpu.sync_copy(dst_hbm.at[wid], idx_v)
    plsc.subcore_barrier()

    def body(j, carry):
        pltpu.sync_copy(ones_v, acc_sh.at[idx_v.at[j]], add=True)
        return carry

    lax.fori_loop(0, CPT, body, 0)
    plsc.subcore_barrier()
    pltpu.sync_copy(acc_sh.at[pl.ds(s * DEG_PT, DEG_PT)],
                    out_hbm.at[c, pl.ds(s * DEG_PT, DEG_PT)])


@functools.cache
def _deg_call():
    return pl.kernel(
        _deg_body,
        out_type=jax.ShapeDtypeStruct((NC, DEG_LEN), jnp.float32),
        mesh=_mesh(),
        scratch_types=[
            pltpu.VMEM((CPT, CH), jnp.int32),
            pltpu.VMEM((CH,), jnp.float32),
            pltpu.VMEM((DEG_PT,), jnp.float32),
            pltpu.VMEM_SHARED((DEG_LEN,), jnp.float32),
        ],
    )


def _agg_body(hs_hbm, src_hbm, dst_hbm, out_hbm, srcv, dstv, rows_v, acc_sh,
              sem):
    c = lax.axis_index("c")
    s = lax.axis_index("s")
    wid = s * NC + c

    def zrow(j, carry):
        for k in range(H // 16):
            rows_v[j, pl.ds(k * 16, 16)] = jnp.zeros((16,), jnp.float32)
        return carry

    lax.fori_loop(0, CH, zrow, 0)
    base = s * ROWS_INIT
    off = 0
    for nblk in (128, 128, 128, 128, ROWS_INIT - 4 * 128):
        pltpu.sync_copy(rows_v.at[pl.ds(0, nblk)],
                        acc_sh.at[pl.ds(base + off, nblk)])
        off += nblk
    pltpu.sync_copy(src_hbm.at[wid], srcv)
    pltpu.sync_copy(dst_hbm.at[wid], dstv)
    plsc.subcore_barrier()

    def body(j, carry):
        pltpu.async_copy(hs_hbm.at[srcv.at[j]], rows_v, sem).wait()
        pltpu.sync_copy(rows_v, acc_sh.at[dstv.at[j]], add=True)
        return carry

    lax.fori_loop(0, CPT, body, 0)
    plsc.subcore_barrier()
    pltpu.sync_copy(acc_sh.at[pl.ds(s * ROWS_OUT, ROWS_OUT)],
                    out_hbm.at[c, pl.ds(s * ROWS_OUT, ROWS_OUT)])

    @pl.when(s == NS - 1)
    def _tail():
        pltpu.sync_copy(acc_sh.at[pl.ds(NS * ROWS_OUT, N - NS * ROWS_OUT)],
                        out_hbm.at[c, pl.ds(NS * ROWS_OUT, N - NS * ROWS_OUT)])


@functools.cache
def _agg_call():
    return pl.kernel(
        _agg_body,
        out_type=jax.ShapeDtypeStruct((NC, N, H), jnp.float32),
        mesh=_mesh(),
        scratch_types=[
            pltpu.VMEM((CPT, CH), jnp.int32),
            pltpu.VMEM((CPT, CH), jnp.int32),
            pltpu.VMEM((CH, H), jnp.float32),
            pltpu.VMEM_SHARED((ACC_ROWS, H), jnp.float32),
            pltpu.SemaphoreType.DMA,
        ],
    )


def _prep_body(degp_ref, x_ref, w_ref, hs_ref, dinv_ref):
    deg = degp_ref[0] + degp_ref[1] + 1.0
    dinv = lax.rsqrt(deg)
    dinv_ref[...] = dinv
    hs_ref[...] = dinv * jnp.dot(x_ref[...], w_ref[...],
                                 preferred_element_type=jnp.float32)


def _mid_body(accs_ref, hsp_ref, dinv_ref, b_ref, g_ref, be_ref, m_ref,
              v_ref, w_ref, out_ref):
    dinv = dinv_ref[...]
    agg = dinv * (accs_ref[0] + accs_ref[1] + hsp_ref[...]) + b_ref[...]
    sc = g_ref[...] * lax.rsqrt(v_ref[...] + EPS)
    h = jnp.maximum(agg * sc + (be_ref[...] - m_ref[...] * sc), 0.0)
    out_ref[...] = dinv * jnp.dot(h, w_ref[...],
                                  preferred_element_type=jnp.float32)


def _final_body(accs_ref, hsp_ref, dinv_ref, b_ref, g_ref, be_ref, m_ref,
                v_ref, batch_ref, fw1_ref, fb1_ref, fw2_ref, fb2_ref,
                out_ref):
    dinv = dinv_ref[...]
    agg = dinv * (accs_ref[0] + accs_ref[1] + hsp_ref[...]) + b_ref[...]
    sc = g_ref[...] * lax.rsqrt(v_ref[...] + EPS)
    h = jnp.maximum(agg * sc + (be_ref[...] - m_ref[...] * sc), 0.0)
    gid = lax.broadcasted_iota(jnp.int32, (NG, N), 0)
    mask = jnp.where(gid == batch_ref[...], 1.0, 0.0)
    sums = jnp.dot(mask, h, preferred_element_type=jnp.float32)
    cnt = jnp.sum(mask, axis=1, keepdims=True)
    pooled = sums / jnp.maximum(cnt, 1.0)
    r = jnp.maximum(
        jnp.dot(pooled, fw1_ref[...], preferred_element_type=jnp.float32)
        + fb1_ref[...], 0.0)
    o = (jnp.dot(r, fw2_ref[...], preferred_element_type=jnp.float32)
         + fb2_ref[...])
    mx = jnp.max(o, axis=1, keepdims=True)
    ex = jnp.exp(o - mx)
    out_ref[...] = o - mx - jnp.log(jnp.sum(ex, axis=1, keepdims=True))


def _tc(body, out_shape, *args):
    return pl.pallas_call(body, out_shape=out_shape)(*args)


def kernel(x, edge_index, batch, W1, b1, W2, b2, W3, b3, g1, be1, m1, v1,
           g2, be2, m2, v2, g3, be3, m3, v3, fw1, fb1, fw2, fb2):
    f32 = jnp.float32
    src = edge_index[0]
    dst = edge_index[1]
    pad = EPAD - E
    src3 = jnp.concatenate([src, jnp.zeros((pad,), jnp.int32)]).reshape(
        NW, CPT, CH)
    dst3 = jnp.concatenate([dst, jnp.full((pad,), DUMP, jnp.int32)]).reshape(
        NW, CPT, CH)

    deg_p = _deg_call()(dst3)
    degp = deg_p[:, :N].reshape(NC, N, 1)

    row = lambda a: a.reshape(1, -1)
    hs1, dinv = _tc(
        _prep_body,
        (jax.ShapeDtypeStruct((N, H), f32), jax.ShapeDtypeStruct((N, 1), f32)),
        degp, x, W1)

    accs1 = _agg_call()(hs1, src3, dst3)
    hs2 = _tc(_mid_body, jax.ShapeDtypeStruct((N, H), f32),
              accs1, hs1, dinv, row(b1), row(g1), row(be1), row(m1), row(v1),
              W2)
    accs2 = _agg_call()(hs2, src3, dst3)
    hs3 = _tc(_mid_body, jax.ShapeDtypeStruct((N, H), f32),
              accs2, hs2, dinv, row(b2), row(g2), row(be2), row(m2), row(v2),
              W3)
    accs3 = _agg_call()(hs3, src3, dst3)
    out = _tc(_final_body, jax.ShapeDtypeStruct((NG, 2), f32),
              accs3, hs3, dinv, row(b3), row(g3), row(be3), row(m3), row(v3),
              row(batch.astype(jnp.int32)), fw1, row(fb1), fw2, row(fb2))
    return out
